# Initial kernel scaffold; baseline (speedup 1.0000x reference)
import functools

import jax
import jax.numpy as jnp
from jax import lax
from jax.experimental import pallas as pl
from jax.experimental.pallas import tpu as pltpu

_TOPK = 100
_NMS_IOU = 0.7
_PRE_TOPK = 10000
_CHUNK = 128
_NCHUNKS = (_PRE_TOPK + _CHUNK - 1) // _CHUNK
_NPAD = _NCHUNKS * _CHUNK


def _nms_body(boxes_ref, out_ref,
              kx1, ky1, kx2, ky2, karea, supp_ref):
    iota_c = lax.broadcasted_iota(jnp.int32, (1, _CHUNK), 1)
    iota_r = lax.broadcasted_iota(jnp.int32, (_CHUNK, 1), 0)

    def chunk_body(carry):
        chunk_id, kc0, sc0 = carry
        base = chunk_id * _CHUNK
        x1 = boxes_ref[0, 0:1, pl.ds(base, _CHUNK)]
        y1 = boxes_ref[0, 1:2, pl.ds(base, _CHUNK)]
        x2 = boxes_ref[0, 2:3, pl.ds(base, _CHUNK)]
        y2 = boxes_ref[0, 3:4, pl.ds(base, _CHUNK)]
        area = (x2 - x1) * (y2 - y1)

        xx1 = jnp.maximum(x1, kx1[:, :])
        yy1 = jnp.maximum(y1, ky1[:, :])
        xx2 = jnp.minimum(x2, kx2[:, :])
        yy2 = jnp.minimum(y2, ky2[:, :])
        inter = jnp.maximum(xx2 - xx1, 0.0) * jnp.maximum(yy2 - yy1, 0.0)
        iou = inter / (karea[:, :] + area - inter + 1e-12)
        kvalid = iota_r < kc0
        supp_by_kept = jnp.sum(
            jnp.where((iou > _NMS_IOU) & kvalid, 1, 0), axis=0, keepdims=True)
        surv0 = (supp_by_kept == 0) & (base + iota_c < _PRE_TOPK)

        def step(j, c):
            surv, kc, sc = c
            sel = iota_c == j
            selfv = sel.astype(jnp.float32)
            sj = jnp.sum(jnp.where(sel, surv.astype(jnp.int32), 0))
            validj = base + j < _PRE_TOPK
            is_keep = (sj > 0) & (kc < _TOPK)
            xj1 = jnp.sum(x1 * selfv); yj1 = jnp.sum(y1 * selfv)
            xj2 = jnp.sum(x2 * selfv); yj2 = jnp.sum(y2 * selfv)
            aj = jnp.sum(area * selfv)
            sxx1 = jnp.maximum(xj1, x1); syy1 = jnp.maximum(yj1, y1)
            sxx2 = jnp.minimum(xj2, x2); syy2 = jnp.minimum(yj2, y2)
            sint = jnp.maximum(sxx2 - sxx1, 0.0) * jnp.maximum(syy2 - syy1, 0.0)
            siou = sint / (aj + area - sint + 1e-12)
            suppress = (siou > _NMS_IOU) & (iota_c > j)
            surv2 = jnp.where(is_keep, surv & ~suppress, surv)
            keepsel = is_keep & (iota_c == kc)
            out_ref[0:1, :] = jnp.where(keepsel, base + j, out_ref[0:1, :])
            rsel = is_keep & (iota_r == kc)
            kx1[:, :] = jnp.where(rsel, xj1, kx1[:, :])
            ky1[:, :] = jnp.where(rsel, yj1, ky1[:, :])
            kx2[:, :] = jnp.where(rsel, xj2, kx2[:, :])
            ky2[:, :] = jnp.where(rsel, yj2, ky2[:, :])
            karea[:, :] = jnp.where(rsel, aj, karea[:, :])
            is_supp = validj & (sj == 0)
            suppsel = is_supp & (iota_c == sc)
            supp_ref[0:1, :] = jnp.where(suppsel, base + j,
                                         supp_ref[0:1, :])
            return (surv2,
                    kc + is_keep.astype(jnp.int32),
                    jnp.minimum(sc + is_supp.astype(jnp.int32), _CHUNK - 1))

        _, kc, sc = lax.fori_loop(0, _CHUNK, step, (surv0, kc0, sc0))
        return chunk_id + 1, kc, sc

    def chunk_cond(carry):
        chunk_id, kc, _ = carry
        return (chunk_id < _NCHUNKS) & (kc < _TOPK)

    _, kc, _ = lax.while_loop(chunk_cond, chunk_body, (0, 0, 0))

    @pl.when(kc < _TOPK)
    def _pad():
        shift = ((iota_r - iota_c) == -kc).astype(jnp.float32)
        suppf = supp_ref[0:1, :].astype(jnp.float32)
        shifted = jax.lax.dot_general(
            suppf, shift, (((1,), (0,)), ((), ())),
            preferred_element_type=jnp.float32)
        filled = shifted.astype(jnp.int32)
        out_ref[0:1, :] = jnp.where(iota_c < kc, out_ref[0:1, :], filled)


@jax.jit
def kernel(pred_logits, pred_boxes, target_sizes):
    bs, nq, nc = pred_logits.shape
    prob = jax.nn.sigmoid(pred_logits).reshape(bs, nq * nc)
    top_scores, pre_idx = lax.top_k(prob, _PRE_TOPK)
    q = pre_idx // nc
    lbl = pre_idx % nc

    cx = pred_boxes[..., 0]; cy = pred_boxes[..., 1]
    w = pred_boxes[..., 2]; h = pred_boxes[..., 3]
    xyxy = jnp.stack([cx - 0.5 * w, cy - 0.5 * h,
                      cx + 0.5 * w, cy + 0.5 * h], axis=-1)
    img_h = target_sizes[:, 0].astype(jnp.float32)
    img_w = target_sizes[:, 1].astype(jnp.float32)
    scale = jnp.stack([img_w, img_h, img_w, img_h], axis=1)
    xyxy = xyxy * scale[:, None, :]

    cand = jnp.take_along_axis(xyxy, q[..., None], axis=1)
    max_coord = cand.max(axis=(1, 2))
    off = lbl.astype(jnp.float32) * (max_coord[:, None] + 1.0)
    boxes_off = cand + off[..., None]

    bt = jnp.swapaxes(boxes_off, 1, 2)
    bt = jnp.pad(bt, ((0, 0), (0, 0), (0, _NPAD - _PRE_TOPK)))

    keep_pos = pl.pallas_call(
        _nms_body,
        grid=(bs,),
        in_specs=[pl.BlockSpec((1, 4, _NPAD), lambda i: (i, 0, 0))],
        out_specs=pl.BlockSpec((1, _CHUNK), lambda i: (i, 0)),
        out_shape=jax.ShapeDtypeStruct((bs, _CHUNK), jnp.int32),
        scratch_shapes=[
            pltpu.VMEM((_CHUNK, 1), jnp.float32),
            pltpu.VMEM((_CHUNK, 1), jnp.float32),
            pltpu.VMEM((_CHUNK, 1), jnp.float32),
            pltpu.VMEM((_CHUNK, 1), jnp.float32),
            pltpu.VMEM((_CHUNK, 1), jnp.float32),
            pltpu.VMEM((1, _CHUNK), jnp.int32),
        ],
    )(bt)

    keep = keep_pos[:, :_TOPK]
    scores_out = jnp.take_along_axis(top_scores, keep, axis=1)
    labels_out = jnp.take_along_axis(lbl, keep, axis=1)
    boxes_out = jnp.take_along_axis(cand, keep[..., None], axis=1)
    return scores_out, labels_out, boxes_out


# speedup vs baseline: 95.6407x; 95.6407x over previous
import functools

import jax
import jax.numpy as jnp
from jax import lax
from jax.experimental import pallas as pl
from jax.experimental.pallas import tpu as pltpu

_TOPK = 100
_NMS_IOU = 0.7
_PRE_TOPK = 10000
_CHUNK = 128
_NCHUNKS = (_PRE_TOPK + _CHUNK - 1) // _CHUNK
_NPAD = _NCHUNKS * _CHUNK


def _nms_body(boxes_ref, out_ref,
              kx1, ky1, kx2, ky2, karea, supp_ref):
    iota_c = lax.broadcasted_iota(jnp.int32, (1, _CHUNK), 1)
    iota_r = lax.broadcasted_iota(jnp.int32, (_CHUNK, 1), 0)

    def chunk_body(carry):
        chunk_id, kc0, sc0 = carry
        base = chunk_id * _CHUNK
        x1 = boxes_ref[0, 0:1, pl.ds(base, _CHUNK)]
        y1 = boxes_ref[0, 1:2, pl.ds(base, _CHUNK)]
        x2 = boxes_ref[0, 2:3, pl.ds(base, _CHUNK)]
        y2 = boxes_ref[0, 3:4, pl.ds(base, _CHUNK)]
        area = (x2 - x1) * (y2 - y1)

        xx1 = jnp.maximum(x1, kx1[:, :])
        yy1 = jnp.maximum(y1, ky1[:, :])
        xx2 = jnp.minimum(x2, kx2[:, :])
        yy2 = jnp.minimum(y2, ky2[:, :])
        inter = jnp.maximum(xx2 - xx1, 0.0) * jnp.maximum(yy2 - yy1, 0.0)
        iou = inter / (karea[:, :] + area - inter + 1e-12)
        kvalid = iota_r < kc0
        supp_by_kept = jnp.sum(
            jnp.where((iou > _NMS_IOU) & kvalid, 1, 0), axis=0, keepdims=True)
        surv0 = ((supp_by_kept == 0) &
                 (base + iota_c < _PRE_TOPK)).astype(jnp.int32)

        def step(j, c):
            surv, kc, sc = c
            sel = iota_c == j
            selfv = sel.astype(jnp.float32)
            sj = jnp.sum(jnp.where(sel, surv, 0))
            validj = base + j < _PRE_TOPK
            is_keep = (sj > 0) & (kc < _TOPK)
            xj1 = jnp.sum(x1 * selfv); yj1 = jnp.sum(y1 * selfv)
            xj2 = jnp.sum(x2 * selfv); yj2 = jnp.sum(y2 * selfv)
            aj = jnp.sum(area * selfv)
            sxx1 = jnp.maximum(xj1, x1); syy1 = jnp.maximum(yj1, y1)
            sxx2 = jnp.minimum(xj2, x2); syy2 = jnp.minimum(yj2, y2)
            sint = jnp.maximum(sxx2 - sxx1, 0.0) * jnp.maximum(syy2 - syy1, 0.0)
            siou = sint / (aj + area - sint + 1e-12)
            suppress = (siou > _NMS_IOU) & (iota_c > j)
            surv2 = jnp.where(suppress & is_keep, 0, surv)
            keepsel = is_keep & (iota_c == kc)
            out_ref[0, 0:1, :] = jnp.where(keepsel, base + j, out_ref[0, 0:1, :])
            rsel = is_keep & (iota_r == kc)
            kx1[:, :] = jnp.where(rsel, xj1, kx1[:, :])
            ky1[:, :] = jnp.where(rsel, yj1, ky1[:, :])
            kx2[:, :] = jnp.where(rsel, xj2, kx2[:, :])
            ky2[:, :] = jnp.where(rsel, yj2, ky2[:, :])
            karea[:, :] = jnp.where(rsel, aj, karea[:, :])
            is_supp = validj & (sj == 0)
            suppsel = is_supp & (iota_c == sc)
            supp_ref[0:1, :] = jnp.where(suppsel, base + j,
                                         supp_ref[0:1, :])
            return (surv2,
                    kc + is_keep.astype(jnp.int32),
                    jnp.minimum(sc + is_supp.astype(jnp.int32), _CHUNK - 1))

        _, kc, sc = lax.fori_loop(0, _CHUNK, step, (surv0, kc0, sc0))
        return chunk_id + 1, kc, sc

    def chunk_cond(carry):
        chunk_id, kc, _ = carry
        return (chunk_id < _NCHUNKS) & (kc < _TOPK)

    _, kc, _ = lax.while_loop(chunk_cond, chunk_body, (0, 0, 0))

    @pl.when(kc < _TOPK)
    def _pad():
        shift = ((iota_r - iota_c) == -kc).astype(jnp.float32)
        suppf = supp_ref[0:1, :].astype(jnp.float32)
        shifted = jax.lax.dot_general(
            suppf, shift, (((1,), (0,)), ((), ())),
            preferred_element_type=jnp.float32)
        filled = shifted.astype(jnp.int32)
        out_ref[0, 0:1, :] = jnp.where(iota_c < kc, out_ref[0, 0:1, :], filled)


@jax.jit
def kernel(pred_logits, pred_boxes, target_sizes):
    bs, nq, nc = pred_logits.shape
    prob = jax.nn.sigmoid(pred_logits).reshape(bs, nq * nc)
    top_scores, pre_idx = lax.top_k(prob, _PRE_TOPK)
    q = pre_idx // nc
    lbl = pre_idx % nc

    cx = pred_boxes[..., 0]; cy = pred_boxes[..., 1]
    w = pred_boxes[..., 2]; h = pred_boxes[..., 3]
    xyxy = jnp.stack([cx - 0.5 * w, cy - 0.5 * h,
                      cx + 0.5 * w, cy + 0.5 * h], axis=-1)
    img_h = target_sizes[:, 0].astype(jnp.float32)
    img_w = target_sizes[:, 1].astype(jnp.float32)
    scale = jnp.stack([img_w, img_h, img_w, img_h], axis=1)
    xyxy = xyxy * scale[:, None, :]

    cand = jnp.take_along_axis(xyxy, q[..., None], axis=1)
    max_coord = cand.max(axis=(1, 2))
    off = lbl.astype(jnp.float32) * (max_coord[:, None] + 1.0)
    boxes_off = cand + off[..., None]

    bt = jnp.swapaxes(boxes_off, 1, 2)
    bt = jnp.pad(bt, ((0, 0), (0, 0), (0, _NPAD - _PRE_TOPK)))

    keep_pos = pl.pallas_call(
        _nms_body,
        grid=(bs,),
        in_specs=[pl.BlockSpec((1, 4, _NPAD), lambda i: (i, 0, 0))],
        out_specs=pl.BlockSpec((1, 1, _CHUNK), lambda i: (i, 0, 0)),
        out_shape=jax.ShapeDtypeStruct((bs, 1, _CHUNK), jnp.int32),
        scratch_shapes=[
            pltpu.VMEM((_CHUNK, 1), jnp.float32),
            pltpu.VMEM((_CHUNK, 1), jnp.float32),
            pltpu.VMEM((_CHUNK, 1), jnp.float32),
            pltpu.VMEM((_CHUNK, 1), jnp.float32),
            pltpu.VMEM((_CHUNK, 1), jnp.float32),
            pltpu.VMEM((1, _CHUNK), jnp.int32),
        ],
    )(bt)

    keep = keep_pos[:, 0, :_TOPK]
    scores_out = jnp.take_along_axis(top_scores, keep, axis=1)
    labels_out = jnp.take_along_axis(lbl, keep, axis=1)
    boxes_out = jnp.take_along_axis(cand, keep[..., None], axis=1)
    return scores_out, labels_out, boxes_out


# speedup vs baseline: 599.1520x; 6.2646x over previous
import functools

import jax
import jax.numpy as jnp
from jax import lax
from jax.experimental import pallas as pl
from jax.experimental.pallas import tpu as pltpu
from jax.experimental.pallas import tpu_sc as plsc

_TOPK = 100
_NMS_IOU = 0.7
_PRE_TOPK = 10000
_G = 16
_BS = 8
_NQ = 1000
_NQP = 1024
_NC = 91
_NTOT = _NQ * _NC
_NPT = 91008
_NGR = _NPT // _G
_HB = 8192
_HB2 = 2048
_SBUF = 2048
_TB = 2048
_KMAX = 128
_BATCH_TARGET = 192


def _make_sc():
    mesh = plsc.VectorSubcoreMesh(core_axis_name="c", subcore_axis_name="s")

    @functools.partial(
        pl.kernel, mesh=mesh,
        out_type=jax.ShapeDtypeStruct((_BS, _KMAX), jnp.int32),
        scratch_types=[
            pltpu.VMEM((_NPT,), jnp.float32),
            pltpu.VMEM((_HB,), jnp.int32),
            pltpu.VMEM((_HB2,), jnp.int32),
            pltpu.VMEM((_NQP,), jnp.float32),
            pltpu.VMEM((_NQP,), jnp.float32),
            pltpu.VMEM((_NQP,), jnp.float32),
            pltpu.VMEM((_NQP,), jnp.float32),
            pltpu.VMEM((_NQP,), jnp.float32),
            pltpu.VMEM((_NQP,), jnp.int32),
            pltpu.VMEM((_SBUF,), jnp.int32),
            pltpu.VMEM((_SBUF,), jnp.int32),
            pltpu.VMEM((_SBUF,), jnp.int32),
            pltpu.VMEM((_SBUF,), jnp.int32),
            pltpu.VMEM((_TB,), jnp.int32),
            pltpu.VMEM((_TB,), jnp.int32),
            pltpu.VMEM((_KMAX * _G,), jnp.float32),
            pltpu.VMEM((_KMAX * _G,), jnp.float32),
            pltpu.VMEM((_KMAX * _G,), jnp.float32),
            pltpu.VMEM((_KMAX * _G,), jnp.float32),
            pltpu.VMEM((_KMAX * _G,), jnp.float32),
            pltpu.VMEM((_KMAX,), jnp.int32),
            pltpu.VMEM((_KMAX,), jnp.int32),
        ],
        compiler_params=pltpu.CompilerParams(needs_layout_passes=False),
    )
    def sck(prob_hbm, boxt_hbm, out_hbm,
            scf, hist, hist2, bx1, by1, bx2, by2, mq, qmark,
            sbits, sidx, obits, oidx, tbits, tidx,
            kx1, ky1, kx2, ky2, karea, keepv, suppv):
        nc_ = plsc.get_sparse_core_info().num_cores
        wid = lax.axis_index("s") * nc_ + lax.axis_index("c")

        @pl.when(wid < _BS)
        def _work():
            iota = lax.iota(jnp.int32, _G)
            lane0 = iota == 0
            allm = iota == iota
            zeros = iota * 0
            ones = zeros + 1

            def ext(v, l):
                return jnp.sum(jnp.where(iota == l, v, 0))

            def extf(v, l):
                return jnp.sum(jnp.where(iota == l, v, 0.0))

            def spl(s):
                return jnp.where(allm, s, s)

            pltpu.sync_copy(prob_hbm.at[wid], scf)
            pltpu.sync_copy(boxt_hbm.at[wid, 0], bx1)
            pltpu.sync_copy(boxt_hbm.at[wid, 1], by1)
            pltpu.sync_copy(boxt_hbm.at[wid, 2], bx2)
            pltpu.sync_copy(boxt_hbm.at[wid, 3], by2)

            def mq_body(g, _):
                s = g * _G
                v = jnp.maximum(jnp.maximum(bx1[pl.ds(s, _G)],
                                            by1[pl.ds(s, _G)]),
                                jnp.maximum(bx2[pl.ds(s, _G)],
                                            by2[pl.ds(s, _G)]))
                mq[pl.ds(s, _G)] = v
                qmark[pl.ds(s, _G)] = zeros
                return _
            lax.fori_loop(0, _NQP // _G, mq_body, 0)

            def hz(g, _):
                hist[pl.ds(g * _G, _G)] = zeros
                return _
            lax.fori_loop(0, _HB // _G, hz, 0)

            def h1(g, _):
                bits = plsc.bitcast(scf[pl.ds(g * _G, _G)], jnp.int32)
                b = lax.shift_right_logical(bits, 17)
                plsc.addupdate_scatter(hist, [b], ones, mask=allm)
                return _
            lax.fori_loop(0, _NGR, h1, 0)

            def walk(hist_ref, ngroups, hi_bin, t1, t2):
                def wb(i, c):
                    cum, b1, g1, b2, g2 = c
                    gidx = (hi_bin // _G) - 1 - i
                    v = hist_ref[pl.ds(gidx * _G, _G)]
                    tot = jnp.sum(v)
                    cs = plsc.cumsum(v)
                    suf = cum + tot - cs + v
                    hit1 = (cum < t1) & (cum + tot >= t1)
                    l1 = jnp.max(jnp.where(suf >= t1, iota, jnp.int32(-1)))
                    nb1 = jnp.where(hit1, gidx * _G + l1, b1)
                    ng1 = jnp.where(hit1, ext(suf, l1) - ext(v, l1), g1)
                    hit2 = (cum < t2) & (cum + tot >= t2)
                    l2 = jnp.max(jnp.where(suf >= t2, iota, jnp.int32(-1)))
                    nb2 = jnp.where(hit2, gidx * _G + l2, b2)
                    ng2 = jnp.where(hit2, ext(suf, l2) - ext(v, l2), g2)
                    return cum + tot, nb1, ng1, nb2, ng2
                init = (jnp.int32(0), jnp.int32(-1), jnp.int32(0),
                        jnp.int32(-1), jnp.int32(0))
                return lax.fori_loop(0, ngroups, wb, init)

            _, sb_bin, sb_gt, b10k, b10k_gt = walk(
                hist, _HB // _G, _HB, _BATCH_TARGET, _PRE_TOPK)
            rank = _PRE_TOPK - b10k_gt

            lo_bin0 = jnp.maximum(sb_bin, b10k + 1)
            hi_bits0 = jnp.int32(0x7FFFFFFF)
            lo_bits0 = lo_bin0 * 131072

            def compact(lo_bits, hi_bits, with_tau):
                def c2(g, c):
                    ns, nt = c
                    bits = plsc.bitcast(scf[pl.ds(g * _G, _G)], jnp.int32)
                    gi = g * _G + iota
                    ms = (bits >= lo_bits) & (bits < hi_bits)
                    cs = plsc.cumsum(jnp.where(ms, 1, 0))
                    pos = ns + cs - 1
                    okm = ms & (pos < _SBUF)
                    plsc.store_scatter(sbits, [pos], bits, mask=okm)
                    plsc.store_scatter(sidx, [pos], gi, mask=okm)
                    ns = jnp.minimum(ns + ext(cs, _G - 1), _SBUF)
                    if with_tau:
                        b = lax.shift_right_logical(bits, 17)
                        mt = b == b10k
                        ct = plsc.cumsum(jnp.where(mt, 1, 0))
                        post = nt + ct - 1
                        okt = mt & (post < _TB)
                        plsc.store_scatter(tbits, [post], bits, mask=okt)
                        plsc.store_scatter(tidx, [post], gi, mask=okt)
                        nt = jnp.minimum(nt + ext(ct, _G - 1), _TB)
                        mh = b > b10k
                        q = gi // _NC
                        plsc.addupdate_scatter(qmark, [q], ones, mask=mh)
                    return ns, nt
                return lax.fori_loop(0, _NGR, c2, (jnp.int32(0),
                                                   jnp.int32(0)))

            ns0, nt = compact(lo_bits0, hi_bits0, True)

            def h2z(g, _):
                hist2[pl.ds(g * _G, _G)] = zeros
                return _
            lax.fori_loop(0, _HB2 // _G, h2z, 0)
            ntg = (nt + _G - 1) // _G

            def h2(g, c):
                bits = tbits[pl.ds(g * _G, _G)]
                valid = (g * _G + iota) < nt
                d2 = lax.shift_right_logical(bits, 6) & 0x7FF
                plsc.addupdate_scatter(hist2, [d2], ones, mask=valid)
                return c
            lax.fori_loop(0, ntg, h2, 0)
            _, d2s, d2gt, _, _ = walk(hist2, _HB2 // _G, _HB2, rank, 999999)
            rank2 = rank - d2gt

            def h3z(g, _):
                hist2[pl.ds(g * _G, _G)] = zeros
                return _
            lax.fori_loop(0, 4, h3z, 0)

            def h3(g, c):
                bits = tbits[pl.ds(g * _G, _G)]
                valid = ((g * _G + iota) < nt) & \
                    ((lax.shift_right_logical(bits, 6) & 0x7FF) == d2s)
                d3 = bits & 0x3F
                plsc.addupdate_scatter(hist2, [d3], ones, mask=valid)
                return c
            lax.fori_loop(0, ntg, h3, 0)
            _, d3s, d3gt, _, _ = walk(hist2, 4, 64, rank2, 999999)
            tau_bits = b10k * 131072 + d2s * 64 + d3s
            need = _PRE_TOPK - (b10k_gt + d2gt + d3gt)

            def markt(g, neq):
                bits = tbits[pl.ds(g * _G, _G)]
                gi = tidx[pl.ds(g * _G, _G)]
                valid = (g * _G + iota) < nt
                eq = valid & (bits == tau_bits)
                cs = plsc.cumsum(jnp.where(eq, 1, 0))
                mark = valid & ((bits > tau_bits)
                                | (eq & ((neq + cs) <= need)))
                q = gi // _NC
                plsc.addupdate_scatter(qmark, [q], ones, mask=mark)
                return neq + ext(cs, _G - 1)
            lax.fori_loop(0, ntg, markt, jnp.int32(0))

            def mx(g, m):
                s = g * _G
                v = jnp.where(qmark[pl.ds(s, _G)] > 0, mq[pl.ds(s, _G)],
                              jnp.float32(-3.0e38))
                return jnp.maximum(m, jnp.max(v))
            max_coord = lax.fori_loop(0, _NQP // _G, mx, jnp.float32(-3.0e38))
            offsc = max_coord + 1.0

            def sort_batch(ns):
                def sb(i, _):
                    grp = (i // _G) * _G
                    ib = ext(sbits[pl.ds(grp, _G)], i - grp)

                    def cnt(j, a):
                        v = sbits[pl.ds(j * _G, _G)]
                        jj = j * _G + iota
                        before = (v > ib) | ((v == ib) & (jj < i))
                        before = before & (jj < ns)
                        return a + jnp.sum(jnp.where(before, 1, 0))
                    r = lax.fori_loop(0, (ns + _G - 1) // _G, cnt,
                                      jnp.int32(0))
                    plsc.store_scatter(obits, [spl(r)], spl(ib),
                                       mask=lane0)
                    plsc.store_scatter(
                        oidx, [spl(r)],
                        spl(ext(sidx[pl.ds(grp, _G)], i - grp)), mask=lane0)
                    return _
                lax.fori_loop(0, ns, sb, 0)

            def nms_batch(ns, kc, sc_):
                ng = (ns + _G - 1) // _G

                def group_body(carry):
                    g, kc, sc_ = carry
                    base = g * _G
                    inb0 = (base + iota) < ns
                    gi = jnp.where(inb0, oidx[pl.ds(base, _G)], 0)
                    q = gi // _NC
                    lbl = gi - q * _NC
                    off = lbl.astype(jnp.float32) * offsc
                    gx1 = plsc.load_gather(bx1, [q]) + off
                    gy1 = plsc.load_gather(by1, [q]) + off
                    gx2 = plsc.load_gather(bx2, [q]) + off
                    gy2 = plsc.load_gather(by2, [q]) + off
                    garea = (gx2 - gx1) * (gy2 - gy1)
                    inb = inb0

                    def vs_kept(k, surv):
                        row = k * _G
                        xx1 = jnp.maximum(kx1[pl.ds(row, _G)], gx1)
                        yy1 = jnp.maximum(ky1[pl.ds(row, _G)], gy1)
                        xx2 = jnp.minimum(kx2[pl.ds(row, _G)], gx2)
                        yy2 = jnp.minimum(ky2[pl.ds(row, _G)], gy2)
                        inter = (jnp.maximum(xx2 - xx1, 0.0)
                                 * jnp.maximum(yy2 - yy1, 0.0))
                        iou = inter / (karea[pl.ds(row, _G)] + garea
                                       - inter + 1e-12)
                        return jnp.where(iou > _NMS_IOU, 0, surv)

                    surv0 = jnp.where(inb, 1, 0)
                    surv0 = lax.fori_loop(0, kc, vs_kept, surv0)

                    def lane_body(l, c):
                        surv, kc, sc_ = c
                        sl = ext(surv, l)
                        validj = ext(jnp.where(inb, 1, 0), l) > 0
                        is_keep = (sl > 0) & (kc < _TOPK)
                        gidx = ext(gi, l)
                        xj1 = extf(gx1, l); yj1 = extf(gy1, l)
                        xj2 = extf(gx2, l); yj2 = extf(gy2, l)
                        aj = extf(garea, l)
                        mk = is_keep & allm
                        row = kc * _G
                        plsc.store_scatter(kx1, [row + iota], spl(xj1),
                                           mask=mk)
                        plsc.store_scatter(ky1, [row + iota], spl(yj1),
                                           mask=mk)
                        plsc.store_scatter(kx2, [row + iota], spl(xj2),
                                           mask=mk)
                        plsc.store_scatter(ky2, [row + iota], spl(yj2),
                                           mask=mk)
                        plsc.store_scatter(karea, [row + iota], spl(aj),
                                           mask=mk)
                        plsc.store_scatter(keepv, [spl(kc)], spl(gidx),
                                           mask=is_keep & lane0)
                        sxx1 = jnp.maximum(spl(xj1), gx1)
                        syy1 = jnp.maximum(spl(yj1), gy1)
                        sxx2 = jnp.minimum(spl(xj2), gx2)
                        syy2 = jnp.minimum(spl(yj2), gy2)
                        sint = (jnp.maximum(sxx2 - sxx1, 0.0)
                                * jnp.maximum(syy2 - syy1, 0.0))
                        siou = sint / (spl(aj) + garea - sint + 1e-12)
                        kill = (siou > _NMS_IOU) & (iota > l) & is_keep
                        surv2 = jnp.where(kill, 0, surv)
                        is_supp = validj & (sl == 0)
                        plsc.store_scatter(suppv, [spl(sc_)], spl(gidx),
                                           mask=is_supp & lane0
                                           & (sc_ < _KMAX))
                        return (surv2,
                                kc + jnp.where(is_keep, 1, 0),
                                jnp.minimum(sc_ + jnp.where(is_supp, 1, 0),
                                            _KMAX - 1))

                    _, kc, sc_ = lax.fori_loop(0, _G, lane_body,
                                               (surv0, kc, sc_))
                    return g + 1, kc, sc_

                def group_cond(carry):
                    g, kc, _ = carry
                    return (g < ng) & (kc < _TOPK)

                _, kc, sc_ = lax.while_loop(group_cond, group_body,
                                            (jnp.int32(0), kc, sc_))
                return kc, sc_

            sort_batch(ns0)
            kc, sc_ = nms_batch(ns0, jnp.int32(0), jnp.int32(0))

            def ext_cond(carry):
                kc, sc_, lo_bits, _hi = carry
                return (kc < _TOPK) & (lo_bits > tau_bits)

            def ext_body(carry):
                kc, sc_, lo_bits, _hi = carry
                hi_bin = lax.shift_right_logical(lo_bits, 17)
                def wb(i, c):
                    cum, nb = c
                    gidx = (hi_bin - 1 - i) // 1
                    v = ext(hist[pl.ds((gidx // _G) * _G, _G)],
                            gidx - (gidx // _G) * _G)
                    hit = (cum < _BATCH_TARGET) & (cum + v >= _BATCH_TARGET)
                    nb2 = jnp.where(hit & (nb < 0), gidx, nb)
                    return cum + v, nb2
                nbins = hi_bin - (b10k + 1)
                cum, nb = lax.fori_loop(0, jnp.maximum(nbins, 0), wb,
                                        (jnp.int32(0), jnp.int32(-1)))
                new_lo = jnp.where(nb < 0, tau_bits, nb * 131072)
                ns, _ = compact(new_lo, lo_bits, False)
                sort_batch(ns)
                kc, sc_ = nms_batch(ns, kc, sc_)
                return kc, sc_, new_lo, lo_bits

            kc, sc_, _, _ = lax.while_loop(
                ext_cond, ext_body, (kc, sc_, lo_bits0, hi_bits0))

            @pl.when(kc < _TOPK)
            def _pad():
                def fill(t, _):
                    j = t - kc
                    val = ext(suppv[pl.ds((j // _G) * _G, _G)], j % _G)
                    plsc.store_scatter(keepv, [spl(t)], spl(val),
                                       mask=lane0)
                    return _
                lax.fori_loop(kc, _TOPK, fill, 0)

            pltpu.sync_copy(keepv, out_hbm.at[wid])

    return sck


_make_sc = functools.cache(_make_sc)


@jax.jit
def kernel(pred_logits, pred_boxes, target_sizes):
    bs, nq, nc = pred_logits.shape
    prob = jax.nn.sigmoid(pred_logits).reshape(bs, nq * nc)
    prob_p = jnp.pad(prob, ((0, 0), (0, _NPT - nq * nc)))

    cx = pred_boxes[..., 0]; cy = pred_boxes[..., 1]
    w = pred_boxes[..., 2]; h = pred_boxes[..., 3]
    img_h = target_sizes[:, 0].astype(jnp.float32)
    img_w = target_sizes[:, 1].astype(jnp.float32)
    x1 = (cx - 0.5 * w) * img_w[:, None]
    y1 = (cy - 0.5 * h) * img_h[:, None]
    x2 = (cx + 0.5 * w) * img_w[:, None]
    y2 = (cy + 0.5 * h) * img_h[:, None]
    boxt = jnp.stack([x1, y1, x2, y2], axis=1)
    boxt = jnp.pad(boxt, ((0, 0), (0, 0), (0, _NQP - nq)))

    keep_gidx = _make_sc()(prob_p, boxt)

    keep = keep_gidx[:, :_TOPK]
    scores_out = jnp.take_along_axis(prob, keep, axis=1)
    labels_out = keep % nc
    q = keep // nc
    xy = jnp.stack([x1, y1, x2, y2], axis=-1)
    boxes_out = jnp.take_along_axis(xy, q[..., None], axis=1)
    return scores_out, labels_out, boxes_out


# speedup vs baseline: 609.4042x; 1.0171x over previous
import functools

import jax
import jax.numpy as jnp
from jax import lax
from jax.experimental import pallas as pl
from jax.experimental.pallas import tpu as pltpu
from jax.experimental.pallas import tpu_sc as plsc

_TOPK = 100
_NMS_IOU = 0.7
_PRE_TOPK = 10000
_G = 16
_BS = 8
_NQ = 1000
_NQP = 1024
_NC = 91
_NTOT = _NQ * _NC
_NPT = 91008
_NGR = _NPT // _G
_HB = 8192
_HB2 = 2048
_SBUF = 2048
_TB = 2048
_KMAX = 128
_BATCH_TARGET = 192


def _make_sc():
    mesh = plsc.VectorSubcoreMesh(core_axis_name="c", subcore_axis_name="s")

    @functools.partial(
        pl.kernel, mesh=mesh,
        out_type=jax.ShapeDtypeStruct((_BS, _KMAX), jnp.int32),
        scratch_types=[
            pltpu.VMEM((_NPT,), jnp.float32),
            pltpu.VMEM((_HB,), jnp.int32),
            pltpu.VMEM((_HB2,), jnp.int32),
            pltpu.VMEM((_NQP,), jnp.float32),
            pltpu.VMEM((_NQP,), jnp.float32),
            pltpu.VMEM((_NQP,), jnp.float32),
            pltpu.VMEM((_NQP,), jnp.float32),
            pltpu.VMEM((_NQP,), jnp.float32),
            pltpu.VMEM((_NQP,), jnp.int32),
            pltpu.VMEM((_SBUF,), jnp.int32),
            pltpu.VMEM((_SBUF,), jnp.int32),
            pltpu.VMEM((_SBUF,), jnp.int32),
            pltpu.VMEM((_SBUF,), jnp.int32),
            pltpu.VMEM((_TB,), jnp.int32),
            pltpu.VMEM((_TB,), jnp.int32),
            pltpu.VMEM((_KMAX * _G,), jnp.float32),
            pltpu.VMEM((_KMAX * _G,), jnp.float32),
            pltpu.VMEM((_KMAX * _G,), jnp.float32),
            pltpu.VMEM((_KMAX * _G,), jnp.float32),
            pltpu.VMEM((_KMAX * _G,), jnp.float32),
            pltpu.VMEM((_KMAX,), jnp.int32),
            pltpu.VMEM((_KMAX,), jnp.int32),
        ],
        compiler_params=pltpu.CompilerParams(needs_layout_passes=False),
    )
    def sck(prob_hbm, boxt_hbm, out_hbm,
            scf, hist, hist2, bx1, by1, bx2, by2, mq, qmark,
            sbits, sidx, obits, oidx, tbits, tidx,
            kx1, ky1, kx2, ky2, karea, keepv, suppv):
        nc_ = plsc.get_sparse_core_info().num_cores
        wid = lax.axis_index("s") * nc_ + lax.axis_index("c")

        @pl.when(wid < _BS)
        def _work():
            iota = lax.iota(jnp.int32, _G)
            lane0 = iota == 0
            allm = iota == iota
            zeros = iota * 0
            ones = zeros + 1

            def ext(v, l):
                return jnp.sum(jnp.where(iota == l, v, 0))

            def extf(v, l):
                return jnp.sum(jnp.where(iota == l, v, 0.0))

            def spl(s):
                return jnp.where(allm, s, s)

            pltpu.sync_copy(prob_hbm.at[wid], scf)
            pltpu.sync_copy(boxt_hbm.at[wid, 0], bx1)
            pltpu.sync_copy(boxt_hbm.at[wid, 1], by1)
            pltpu.sync_copy(boxt_hbm.at[wid, 2], bx2)
            pltpu.sync_copy(boxt_hbm.at[wid, 3], by2)

            def mq_body(g, _):
                s = g * _G
                v = jnp.maximum(jnp.maximum(bx1[pl.ds(s, _G)],
                                            by1[pl.ds(s, _G)]),
                                jnp.maximum(bx2[pl.ds(s, _G)],
                                            by2[pl.ds(s, _G)]))
                mq[pl.ds(s, _G)] = v
                qmark[pl.ds(s, _G)] = zeros
                return _
            lax.fori_loop(0, _NQP // _G, mq_body, 0)

            def hz(g, _):
                hist[pl.ds(g * _G, _G)] = zeros
                return _
            lax.fori_loop(0, _HB // _G, hz, 0)

            def h1(g, _):
                for u in range(4):
                    s = (g * 4 + u) * _G
                    bits = plsc.bitcast(scf[pl.ds(s, _G)], jnp.int32)
                    b = lax.shift_right_logical(bits, 17)
                    plsc.addupdate_scatter(hist, [b], ones, mask=allm)
                return _
            lax.fori_loop(0, _NGR // 4, h1, 0)

            def walk(hist_ref, ngroups, hi_bin, t1, t2):
                def wb(i, c):
                    cum, b1, g1, b2, g2 = c
                    gidx = (hi_bin // _G) - 1 - i
                    v = hist_ref[pl.ds(gidx * _G, _G)]
                    tot = jnp.sum(v)
                    cs = plsc.cumsum(v)
                    suf = cum + tot - cs + v
                    hit1 = (cum < t1) & (cum + tot >= t1)
                    l1 = jnp.max(jnp.where(suf >= t1, iota, jnp.int32(-1)))
                    nb1 = jnp.where(hit1, gidx * _G + l1, b1)
                    ng1 = jnp.where(hit1, ext(suf, l1) - ext(v, l1), g1)
                    hit2 = (cum < t2) & (cum + tot >= t2)
                    l2 = jnp.max(jnp.where(suf >= t2, iota, jnp.int32(-1)))
                    nb2 = jnp.where(hit2, gidx * _G + l2, b2)
                    ng2 = jnp.where(hit2, ext(suf, l2) - ext(v, l2), g2)
                    return cum + tot, nb1, ng1, nb2, ng2
                init = (jnp.int32(0), jnp.int32(-1), jnp.int32(0),
                        jnp.int32(-1), jnp.int32(0))
                return lax.fori_loop(0, ngroups, wb, init)

            _, sb_bin, sb_gt, b10k, b10k_gt = walk(
                hist, _HB // _G, _HB, _BATCH_TARGET, _PRE_TOPK)
            rank = _PRE_TOPK - b10k_gt

            lo_bin0 = jnp.maximum(sb_bin, b10k + 1)
            hi_bits0 = jnp.int32(0x7FFFFFFF)
            lo_bits0 = lo_bin0 * 131072

            def compact(lo_bits, hi_bits, with_tau):
                def c2(g, c):
                    ns, nt = c
                    for u in range(4):
                        s = (g * 4 + u) * _G
                        bits = plsc.bitcast(scf[pl.ds(s, _G)], jnp.int32)
                        gi = s + iota
                        ms = (bits >= lo_bits) & (bits < hi_bits)
                        cs = plsc.cumsum(jnp.where(ms, 1, 0))
                        pos = ns + cs - 1
                        okm = ms & (pos < _SBUF)
                        plsc.store_scatter(sbits, [pos], bits, mask=okm)
                        plsc.store_scatter(sidx, [pos], gi, mask=okm)
                        ns = jnp.minimum(ns + ext(cs, _G - 1), _SBUF)
                        if with_tau:
                            b = lax.shift_right_logical(bits, 17)
                            mt = b == b10k
                            ct = plsc.cumsum(jnp.where(mt, 1, 0))
                            post = nt + ct - 1
                            okt = mt & (post < _TB)
                            plsc.store_scatter(tbits, [post], bits, mask=okt)
                            plsc.store_scatter(tidx, [post], gi, mask=okt)
                            nt = jnp.minimum(nt + ext(ct, _G - 1), _TB)
                            mh = b > b10k
                            q = gi // _NC
                            plsc.addupdate_scatter(qmark, [q], ones, mask=mh)
                    return ns, nt
                return lax.fori_loop(0, _NGR // 4, c2, (jnp.int32(0),
                                                        jnp.int32(0)))

            ns0, nt = compact(lo_bits0, hi_bits0, True)

            def h2z(g, _):
                hist2[pl.ds(g * _G, _G)] = zeros
                return _
            lax.fori_loop(0, _HB2 // _G, h2z, 0)
            ntg = (nt + _G - 1) // _G

            def h2(g, c):
                bits = tbits[pl.ds(g * _G, _G)]
                valid = (g * _G + iota) < nt
                d2 = lax.shift_right_logical(bits, 6) & 0x7FF
                plsc.addupdate_scatter(hist2, [d2], ones, mask=valid)
                return c
            lax.fori_loop(0, ntg, h2, 0)
            _, d2s, d2gt, _, _ = walk(hist2, _HB2 // _G, _HB2, rank, 999999)
            rank2 = rank - d2gt

            def h3z(g, _):
                hist2[pl.ds(g * _G, _G)] = zeros
                return _
            lax.fori_loop(0, 4, h3z, 0)

            def h3(g, c):
                bits = tbits[pl.ds(g * _G, _G)]
                valid = ((g * _G + iota) < nt) & \
                    ((lax.shift_right_logical(bits, 6) & 0x7FF) == d2s)
                d3 = bits & 0x3F
                plsc.addupdate_scatter(hist2, [d3], ones, mask=valid)
                return c
            lax.fori_loop(0, ntg, h3, 0)
            _, d3s, d3gt, _, _ = walk(hist2, 4, 64, rank2, 999999)
            tau_bits = b10k * 131072 + d2s * 64 + d3s
            need = _PRE_TOPK - (b10k_gt + d2gt + d3gt)

            def markt(g, neq):
                bits = tbits[pl.ds(g * _G, _G)]
                gi = tidx[pl.ds(g * _G, _G)]
                valid = (g * _G + iota) < nt
                eq = valid & (bits == tau_bits)
                cs = plsc.cumsum(jnp.where(eq, 1, 0))
                mark = valid & ((bits > tau_bits)
                                | (eq & ((neq + cs) <= need)))
                q = gi // _NC
                plsc.addupdate_scatter(qmark, [q], ones, mask=mark)
                return neq + ext(cs, _G - 1)
            lax.fori_loop(0, ntg, markt, jnp.int32(0))

            def mx(g, m):
                s = g * _G
                v = jnp.where(qmark[pl.ds(s, _G)] > 0, mq[pl.ds(s, _G)],
                              jnp.float32(-3.0e38))
                return jnp.maximum(m, jnp.max(v))
            max_coord = lax.fori_loop(0, _NQP // _G, mx, jnp.float32(-3.0e38))
            offsc = max_coord + 1.0

            def sort_batch(ns):
                def sb(i, _):
                    grp = (i // _G) * _G
                    ib = ext(sbits[pl.ds(grp, _G)], i - grp)

                    def cnt(j, a):
                        v = sbits[pl.ds(j * _G, _G)]
                        jj = j * _G + iota
                        before = (v > ib) | ((v == ib) & (jj < i))
                        before = before & (jj < ns)
                        return a + jnp.sum(jnp.where(before, 1, 0))
                    r = lax.fori_loop(0, (ns + _G - 1) // _G, cnt,
                                      jnp.int32(0))
                    plsc.store_scatter(obits, [spl(r)], spl(ib),
                                       mask=lane0)
                    plsc.store_scatter(
                        oidx, [spl(r)],
                        spl(ext(sidx[pl.ds(grp, _G)], i - grp)), mask=lane0)
                    return _
                lax.fori_loop(0, ns, sb, 0)

            def nms_batch(ns, kc, sc_):
                ng = (ns + _G - 1) // _G

                def group_body(carry):
                    g, kc, sc_ = carry
                    base = g * _G
                    inb0 = (base + iota) < ns
                    gi = jnp.where(inb0, oidx[pl.ds(base, _G)], 0)
                    q = gi // _NC
                    lbl = gi - q * _NC
                    off = lbl.astype(jnp.float32) * offsc
                    gx1 = plsc.load_gather(bx1, [q]) + off
                    gy1 = plsc.load_gather(by1, [q]) + off
                    gx2 = plsc.load_gather(bx2, [q]) + off
                    gy2 = plsc.load_gather(by2, [q]) + off
                    garea = (gx2 - gx1) * (gy2 - gy1)
                    inb = inb0

                    def vs_kept(k, surv):
                        row = k * _G
                        xx1 = jnp.maximum(kx1[pl.ds(row, _G)], gx1)
                        yy1 = jnp.maximum(ky1[pl.ds(row, _G)], gy1)
                        xx2 = jnp.minimum(kx2[pl.ds(row, _G)], gx2)
                        yy2 = jnp.minimum(ky2[pl.ds(row, _G)], gy2)
                        inter = (jnp.maximum(xx2 - xx1, 0.0)
                                 * jnp.maximum(yy2 - yy1, 0.0))
                        iou = inter / (karea[pl.ds(row, _G)] + garea
                                       - inter + 1e-12)
                        return jnp.where(iou > _NMS_IOU, 0, surv)

                    surv0 = jnp.where(inb, 1, 0)
                    surv0 = lax.fori_loop(0, kc, vs_kept, surv0)

                    def lane_body(l, c):
                        surv, kc, sc_ = c
                        sl = ext(surv, l)
                        validj = ext(jnp.where(inb, 1, 0), l) > 0
                        is_keep = (sl > 0) & (kc < _TOPK)
                        gidx = ext(gi, l)
                        xj1 = extf(gx1, l); yj1 = extf(gy1, l)
                        xj2 = extf(gx2, l); yj2 = extf(gy2, l)
                        aj = extf(garea, l)
                        mk = is_keep & allm
                        row = kc * _G
                        plsc.store_scatter(kx1, [row + iota], spl(xj1),
                                           mask=mk)
                        plsc.store_scatter(ky1, [row + iota], spl(yj1),
                                           mask=mk)
                        plsc.store_scatter(kx2, [row + iota], spl(xj2),
                                           mask=mk)
                        plsc.store_scatter(ky2, [row + iota], spl(yj2),
                                           mask=mk)
                        plsc.store_scatter(karea, [row + iota], spl(aj),
                                           mask=mk)
                        plsc.store_scatter(keepv, [spl(kc)], spl(gidx),
                                           mask=is_keep & lane0)
                        sxx1 = jnp.maximum(spl(xj1), gx1)
                        syy1 = jnp.maximum(spl(yj1), gy1)
                        sxx2 = jnp.minimum(spl(xj2), gx2)
                        syy2 = jnp.minimum(spl(yj2), gy2)
                        sint = (jnp.maximum(sxx2 - sxx1, 0.0)
                                * jnp.maximum(syy2 - syy1, 0.0))
                        siou = sint / (spl(aj) + garea - sint + 1e-12)
                        kill = (siou > _NMS_IOU) & (iota > l) & is_keep
                        surv2 = jnp.where(kill, 0, surv)
                        is_supp = validj & (sl == 0)
                        plsc.store_scatter(suppv, [spl(sc_)], spl(gidx),
                                           mask=is_supp & lane0
                                           & (sc_ < _KMAX))
                        return (surv2,
                                kc + jnp.where(is_keep, 1, 0),
                                jnp.minimum(sc_ + jnp.where(is_supp, 1, 0),
                                            _KMAX - 1))

                    _, kc, sc_ = lax.fori_loop(0, _G, lane_body,
                                               (surv0, kc, sc_))
                    return g + 1, kc, sc_

                def group_cond(carry):
                    g, kc, _ = carry
                    return (g < ng) & (kc < _TOPK)

                _, kc, sc_ = lax.while_loop(group_cond, group_body,
                                            (jnp.int32(0), kc, sc_))
                return kc, sc_

            sort_batch(ns0)
            kc, sc_ = nms_batch(ns0, jnp.int32(0), jnp.int32(0))

            def ext_cond(carry):
                kc, sc_, lo_bits, _hi = carry
                return (kc < _TOPK) & (lo_bits > tau_bits)

            def ext_body(carry):
                kc, sc_, lo_bits, _hi = carry
                hi_bin = lax.shift_right_logical(lo_bits, 17)
                def wb(i, c):
                    cum, nb = c
                    gidx = (hi_bin - 1 - i) // 1
                    v = ext(hist[pl.ds((gidx // _G) * _G, _G)],
                            gidx - (gidx // _G) * _G)
                    hit = (cum < _BATCH_TARGET) & (cum + v >= _BATCH_TARGET)
                    nb2 = jnp.where(hit & (nb < 0), gidx, nb)
                    return cum + v, nb2
                nbins = hi_bin - (b10k + 1)
                cum, nb = lax.fori_loop(0, jnp.maximum(nbins, 0), wb,
                                        (jnp.int32(0), jnp.int32(-1)))
                new_lo = jnp.where(nb < 0, tau_bits, nb * 131072)
                ns, _ = compact(new_lo, lo_bits, False)
                sort_batch(ns)
                kc, sc_ = nms_batch(ns, kc, sc_)
                return kc, sc_, new_lo, lo_bits

            kc, sc_, _, _ = lax.while_loop(
                ext_cond, ext_body, (kc, sc_, lo_bits0, hi_bits0))

            @pl.when(kc < _TOPK)
            def _pad():
                def fill(t, _):
                    j = t - kc
                    val = ext(suppv[pl.ds((j // _G) * _G, _G)], j % _G)
                    plsc.store_scatter(keepv, [spl(t)], spl(val),
                                       mask=lane0)
                    return _
                lax.fori_loop(kc, _TOPK, fill, 0)

            pltpu.sync_copy(keepv, out_hbm.at[wid])

    return sck


_make_sc = functools.cache(_make_sc)


@jax.jit
def kernel(pred_logits, pred_boxes, target_sizes):
    bs, nq, nc = pred_logits.shape
    prob = jax.nn.sigmoid(pred_logits).reshape(bs, nq * nc)
    prob_p = jnp.pad(prob, ((0, 0), (0, _NPT - nq * nc)))

    cx = pred_boxes[..., 0]; cy = pred_boxes[..., 1]
    w = pred_boxes[..., 2]; h = pred_boxes[..., 3]
    img_h = target_sizes[:, 0].astype(jnp.float32)
    img_w = target_sizes[:, 1].astype(jnp.float32)
    x1 = (cx - 0.5 * w) * img_w[:, None]
    y1 = (cy - 0.5 * h) * img_h[:, None]
    x2 = (cx + 0.5 * w) * img_w[:, None]
    y2 = (cy + 0.5 * h) * img_h[:, None]
    boxt = jnp.stack([x1, y1, x2, y2], axis=1)
    boxt = jnp.pad(boxt, ((0, 0), (0, 0), (0, _NQP - nq)))

    keep_gidx = _make_sc()(prob_p, boxt)

    keep = keep_gidx[:, :_TOPK]
    scores_out = jnp.take_along_axis(prob, keep, axis=1)
    labels_out = keep % nc
    q = keep // nc
    xy = jnp.stack([x1, y1, x2, y2], axis=-1)
    boxes_out = jnp.take_along_axis(xy, q[..., None], axis=1)
    return scores_out, labels_out, boxes_out


# speedup vs baseline: 680.9902x; 1.1175x over previous
import functools

import jax
import jax.numpy as jnp
from jax import lax
from jax.experimental import pallas as pl
from jax.experimental.pallas import tpu as pltpu
from jax.experimental.pallas import tpu_sc as plsc

_TOPK = 100
_NMS_IOU = 0.7
_PRE_TOPK = 10000
_G = 16
_BS = 8
_NQ = 1000
_NQP = 1024
_NC = 91
_NTOT = _NQ * _NC
_NPT = 91008
_NGR = _NPT // _G
_HB = 8192
_HB2 = 2048
_SBUF = 2048
_TB = 2048
_KMAX = 128
_BATCH_TARGET = 128


def _make_sc():
    mesh = plsc.VectorSubcoreMesh(core_axis_name="c", subcore_axis_name="s")

    @functools.partial(
        pl.kernel, mesh=mesh,
        out_type=jax.ShapeDtypeStruct((_BS, _KMAX), jnp.int32),
        scratch_types=[
            pltpu.VMEM((_NPT,), jnp.float32),
            pltpu.VMEM((_HB,), jnp.int32),
            pltpu.VMEM((_HB2,), jnp.int32),
            pltpu.VMEM((_NQP,), jnp.float32),
            pltpu.VMEM((_NQP,), jnp.float32),
            pltpu.VMEM((_NQP,), jnp.float32),
            pltpu.VMEM((_NQP,), jnp.float32),
            pltpu.VMEM((_NQP,), jnp.float32),
            pltpu.VMEM((_NQP,), jnp.int32),
            pltpu.VMEM((_SBUF,), jnp.int32),
            pltpu.VMEM((_SBUF,), jnp.int32),
            pltpu.VMEM((_SBUF,), jnp.int32),
            pltpu.VMEM((_SBUF,), jnp.int32),
            pltpu.VMEM((_TB,), jnp.int32),
            pltpu.VMEM((_TB,), jnp.int32),
            pltpu.VMEM((_KMAX * _G,), jnp.float32),
            pltpu.VMEM((_KMAX * _G,), jnp.float32),
            pltpu.VMEM((_KMAX * _G,), jnp.float32),
            pltpu.VMEM((_KMAX * _G,), jnp.float32),
            pltpu.VMEM((_KMAX * _G,), jnp.float32),
            pltpu.VMEM((_KMAX,), jnp.int32),
            pltpu.VMEM((_KMAX,), jnp.int32),
        ],
        compiler_params=pltpu.CompilerParams(needs_layout_passes=False),
    )
    def sck(prob_hbm, boxt_hbm, out_hbm,
            scf, hist, hist2, bx1, by1, bx2, by2, mq, qmark,
            sbits, sidx, obits, oidx, tbits, tidx,
            kx1, ky1, kx2, ky2, karea, keepv, suppv):
        nc_ = plsc.get_sparse_core_info().num_cores
        wid = lax.axis_index("s") * nc_ + lax.axis_index("c")

        @pl.when(wid < _BS)
        def _work():
            iota = lax.iota(jnp.int32, _G)
            lane0 = iota == 0
            allm = iota == iota
            zeros = iota * 0
            ones = zeros + 1

            def ext(v, l):
                return jnp.sum(jnp.where(iota == l, v, 0))

            def extf(v, l):
                return jnp.sum(jnp.where(iota == l, v, 0.0))

            def spl(s):
                return jnp.where(allm, s, s)

            pltpu.sync_copy(prob_hbm.at[wid], scf)
            pltpu.sync_copy(boxt_hbm.at[wid, 0], bx1)
            pltpu.sync_copy(boxt_hbm.at[wid, 1], by1)
            pltpu.sync_copy(boxt_hbm.at[wid, 2], bx2)
            pltpu.sync_copy(boxt_hbm.at[wid, 3], by2)

            def mq_body(g, _):
                s = g * _G
                v = jnp.maximum(jnp.maximum(bx1[pl.ds(s, _G)],
                                            by1[pl.ds(s, _G)]),
                                jnp.maximum(bx2[pl.ds(s, _G)],
                                            by2[pl.ds(s, _G)]))
                mq[pl.ds(s, _G)] = v
                qmark[pl.ds(s, _G)] = zeros
                return _
            lax.fori_loop(0, _NQP // _G, mq_body, 0)

            def hz(g, _):
                for u in range(8):
                    hist[pl.ds((g * 8 + u) * _G, _G)] = zeros
                return _
            lax.fori_loop(0, _HB // _G // 8, hz, 0)

            def h1(g, _):
                for u in range(4):
                    s = (g * 4 + u) * _G
                    bits = plsc.bitcast(scf[pl.ds(s, _G)], jnp.int32)
                    b = lax.shift_right_logical(bits, 17)
                    plsc.addupdate_scatter(hist, [b], ones, mask=allm)
                return _
            lax.fori_loop(0, _NGR // 4, h1, 0)

            def walk(hist_ref, ngroups, hi_bin, t1, t2):
                def wcond(c):
                    i, _cum, _b1, _g1, b2, _g2 = c
                    return (i < ngroups) & (b2 < 0)

                def wb(c):
                    i, cum, b1, g1, b2, g2 = c
                    gidx = (hi_bin // _G) - 1 - i
                    v = hist_ref[pl.ds(gidx * _G, _G)]
                    tot = jnp.sum(v)
                    cs = plsc.cumsum(v)
                    suf = cum + tot - cs + v
                    hit1 = (cum < t1) & (cum + tot >= t1)
                    l1 = jnp.max(jnp.where(suf >= t1, iota, jnp.int32(-1)))
                    nb1 = jnp.where(hit1, gidx * _G + l1, b1)
                    ng1 = jnp.where(hit1, ext(suf, l1) - ext(v, l1), g1)
                    hit2 = (cum < t2) & (cum + tot >= t2)
                    l2 = jnp.max(jnp.where(suf >= t2, iota, jnp.int32(-1)))
                    nb2 = jnp.where(hit2, gidx * _G + l2, b2)
                    ng2 = jnp.where(hit2, ext(suf, l2) - ext(v, l2), g2)
                    return i + 1, cum + tot, nb1, ng1, nb2, ng2
                init = (jnp.int32(0), jnp.int32(0), jnp.int32(-1),
                        jnp.int32(0), jnp.int32(-1), jnp.int32(0))
                out = lax.while_loop(wcond, wb, init)
                return out[1:]

            _, sb_bin, sb_gt, b10k, b10k_gt = walk(
                hist, _HB // _G, _HB, _BATCH_TARGET, _PRE_TOPK)
            rank = _PRE_TOPK - b10k_gt

            lo_bin0 = jnp.maximum(sb_bin, b10k + 1)
            hi_bits0 = jnp.int32(0x7FFFFFFF)
            lo_bits0 = lo_bin0 * 131072

            def compact(lo_bits, hi_bits, with_tau):
                def c2(g, c):
                    ns, nt = c
                    for u in range(4):
                        s = (g * 4 + u) * _G
                        bits = plsc.bitcast(scf[pl.ds(s, _G)], jnp.int32)
                        gi = s + iota
                        ms = (bits >= lo_bits) & (bits < hi_bits)
                        cs = plsc.cumsum(jnp.where(ms, 1, 0))
                        pos = ns + cs - 1
                        okm = ms & (pos < _SBUF)
                        plsc.store_scatter(sbits, [pos], bits, mask=okm)
                        plsc.store_scatter(sidx, [pos], gi, mask=okm)
                        ns = jnp.minimum(ns + ext(cs, _G - 1), _SBUF)
                        if with_tau:
                            b = lax.shift_right_logical(bits, 17)
                            mt = b == b10k
                            ct = plsc.cumsum(jnp.where(mt, 1, 0))
                            post = nt + ct - 1
                            okt = mt & (post < _TB)
                            plsc.store_scatter(tbits, [post], bits, mask=okt)
                            plsc.store_scatter(tidx, [post], gi, mask=okt)
                            nt = jnp.minimum(nt + ext(ct, _G - 1), _TB)
                            mh = b > b10k
                            q = gi // _NC
                            plsc.addupdate_scatter(qmark, [q], ones, mask=mh)
                    return ns, nt
                return lax.fori_loop(0, _NGR // 4, c2, (jnp.int32(0),
                                                        jnp.int32(0)))

            ns0, nt = compact(lo_bits0, hi_bits0, True)

            def h2z(g, _):
                hist2[pl.ds(g * _G, _G)] = zeros
                return _
            lax.fori_loop(0, _HB2 // _G, h2z, 0)
            ntg = (nt + _G - 1) // _G

            def h2(g, c):
                bits = tbits[pl.ds(g * _G, _G)]
                valid = (g * _G + iota) < nt
                d2 = lax.shift_right_logical(bits, 6) & 0x7FF
                plsc.addupdate_scatter(hist2, [d2], ones, mask=valid)
                return c
            lax.fori_loop(0, ntg, h2, 0)
            _, d2s, d2gt, _, _ = walk(hist2, _HB2 // _G, _HB2, rank, 999999)
            rank2 = rank - d2gt

            def h3z(g, _):
                hist2[pl.ds(g * _G, _G)] = zeros
                return _
            lax.fori_loop(0, 4, h3z, 0)

            def h3(g, c):
                bits = tbits[pl.ds(g * _G, _G)]
                valid = ((g * _G + iota) < nt) & \
                    ((lax.shift_right_logical(bits, 6) & 0x7FF) == d2s)
                d3 = bits & 0x3F
                plsc.addupdate_scatter(hist2, [d3], ones, mask=valid)
                return c
            lax.fori_loop(0, ntg, h3, 0)
            _, d3s, d3gt, _, _ = walk(hist2, 4, 64, rank2, 999999)
            tau_bits = b10k * 131072 + d2s * 64 + d3s
            need = _PRE_TOPK - (b10k_gt + d2gt + d3gt)

            def markt(g, neq):
                bits = tbits[pl.ds(g * _G, _G)]
                gi = tidx[pl.ds(g * _G, _G)]
                valid = (g * _G + iota) < nt
                eq = valid & (bits == tau_bits)
                cs = plsc.cumsum(jnp.where(eq, 1, 0))
                mark = valid & ((bits > tau_bits)
                                | (eq & ((neq + cs) <= need)))
                q = gi // _NC
                plsc.addupdate_scatter(qmark, [q], ones, mask=mark)
                return neq + ext(cs, _G - 1)
            lax.fori_loop(0, ntg, markt, jnp.int32(0))

            def mx(g, m):
                s = g * _G
                v = jnp.where(qmark[pl.ds(s, _G)] > 0, mq[pl.ds(s, _G)],
                              jnp.float32(-3.0e38))
                return jnp.maximum(m, jnp.max(v))
            max_coord = lax.fori_loop(0, _NQP // _G, mx, jnp.float32(-3.0e38))
            offsc = max_coord + 1.0

            def sort_batch(ns):
                def sb(i, _):
                    grp = (i // _G) * _G
                    ib = ext(sbits[pl.ds(grp, _G)], i - grp)

                    def cnt(j, a):
                        v = sbits[pl.ds(j * _G, _G)]
                        jj = j * _G + iota
                        before = (v > ib) | ((v == ib) & (jj < i))
                        before = before & (jj < ns)
                        return a + jnp.where(before, 1, 0)
                    av = lax.fori_loop(0, (ns + _G - 1) // _G, cnt, zeros)
                    r = jnp.sum(av)
                    plsc.store_scatter(obits, [spl(r)], spl(ib),
                                       mask=lane0)
                    plsc.store_scatter(
                        oidx, [spl(r)],
                        spl(ext(sidx[pl.ds(grp, _G)], i - grp)), mask=lane0)
                    return _
                lax.fori_loop(0, ns, sb, 0)

            def nms_batch(ns, kc, sc_):
                ng = (ns + _G - 1) // _G

                def group_body(carry):
                    g, kc, sc_ = carry
                    base = g * _G
                    inb0 = (base + iota) < ns
                    gi = jnp.where(inb0, oidx[pl.ds(base, _G)], 0)
                    q = gi // _NC
                    lbl = gi - q * _NC
                    off = lbl.astype(jnp.float32) * offsc
                    gx1 = plsc.load_gather(bx1, [q]) + off
                    gy1 = plsc.load_gather(by1, [q]) + off
                    gx2 = plsc.load_gather(bx2, [q]) + off
                    gy2 = plsc.load_gather(by2, [q]) + off
                    garea = (gx2 - gx1) * (gy2 - gy1)
                    inb = inb0

                    def vs_kept(k, surv):
                        row = k * _G
                        xx1 = jnp.maximum(kx1[pl.ds(row, _G)], gx1)
                        yy1 = jnp.maximum(ky1[pl.ds(row, _G)], gy1)
                        xx2 = jnp.minimum(kx2[pl.ds(row, _G)], gx2)
                        yy2 = jnp.minimum(ky2[pl.ds(row, _G)], gy2)
                        inter = (jnp.maximum(xx2 - xx1, 0.0)
                                 * jnp.maximum(yy2 - yy1, 0.0))
                        iou = inter / (karea[pl.ds(row, _G)] + garea
                                       - inter + 1e-12)
                        return jnp.where(iou > _NMS_IOU, 0, surv)

                    surv0 = jnp.where(inb, 1, 0)
                    surv0 = lax.fori_loop(0, kc, vs_kept, surv0)

                    def lane_body(l, c):
                        surv, kc, sc_ = c
                        sl = ext(surv, l)
                        validj = ext(jnp.where(inb, 1, 0), l) > 0
                        is_keep = (sl > 0) & (kc < _TOPK)
                        gidx = ext(gi, l)
                        xj1 = extf(gx1, l); yj1 = extf(gy1, l)
                        xj2 = extf(gx2, l); yj2 = extf(gy2, l)
                        aj = extf(garea, l)
                        mk = is_keep & allm
                        row = kc * _G
                        plsc.store_scatter(kx1, [row + iota], spl(xj1),
                                           mask=mk)
                        plsc.store_scatter(ky1, [row + iota], spl(yj1),
                                           mask=mk)
                        plsc.store_scatter(kx2, [row + iota], spl(xj2),
                                           mask=mk)
                        plsc.store_scatter(ky2, [row + iota], spl(yj2),
                                           mask=mk)
                        plsc.store_scatter(karea, [row + iota], spl(aj),
                                           mask=mk)
                        plsc.store_scatter(keepv, [spl(kc)], spl(gidx),
                                           mask=is_keep & lane0)
                        sxx1 = jnp.maximum(spl(xj1), gx1)
                        syy1 = jnp.maximum(spl(yj1), gy1)
                        sxx2 = jnp.minimum(spl(xj2), gx2)
                        syy2 = jnp.minimum(spl(yj2), gy2)
                        sint = (jnp.maximum(sxx2 - sxx1, 0.0)
                                * jnp.maximum(syy2 - syy1, 0.0))
                        siou = sint / (spl(aj) + garea - sint + 1e-12)
                        kill = (siou > _NMS_IOU) & (iota > l) & is_keep
                        surv2 = jnp.where(kill, 0, surv)
                        is_supp = validj & (sl == 0)
                        plsc.store_scatter(suppv, [spl(sc_)], spl(gidx),
                                           mask=is_supp & lane0
                                           & (sc_ < _KMAX))
                        return (surv2,
                                kc + jnp.where(is_keep, 1, 0),
                                jnp.minimum(sc_ + jnp.where(is_supp, 1, 0),
                                            _KMAX - 1))

                    _, kc, sc_ = lax.fori_loop(0, _G, lane_body,
                                               (surv0, kc, sc_))
                    return g + 1, kc, sc_

                def group_cond(carry):
                    g, kc, _ = carry
                    return (g < ng) & (kc < _TOPK)

                _, kc, sc_ = lax.while_loop(group_cond, group_body,
                                            (jnp.int32(0), kc, sc_))
                return kc, sc_

            sort_batch(ns0)
            kc, sc_ = nms_batch(ns0, jnp.int32(0), jnp.int32(0))

            def ext_cond(carry):
                kc, sc_, lo_bits, _hi = carry
                return (kc < _TOPK) & (lo_bits > tau_bits)

            def ext_body(carry):
                kc, sc_, lo_bits, _hi = carry
                hi_bin = lax.shift_right_logical(lo_bits, 17)
                def wb(i, c):
                    cum, nb = c
                    gidx = (hi_bin - 1 - i) // 1
                    v = ext(hist[pl.ds((gidx // _G) * _G, _G)],
                            gidx - (gidx // _G) * _G)
                    hit = (cum < _BATCH_TARGET) & (cum + v >= _BATCH_TARGET)
                    nb2 = jnp.where(hit & (nb < 0), gidx, nb)
                    return cum + v, nb2
                nbins = hi_bin - (b10k + 1)
                cum, nb = lax.fori_loop(0, jnp.maximum(nbins, 0), wb,
                                        (jnp.int32(0), jnp.int32(-1)))
                new_lo = jnp.where(nb < 0, tau_bits, nb * 131072)
                ns, _ = compact(new_lo, lo_bits, False)
                sort_batch(ns)
                kc, sc_ = nms_batch(ns, kc, sc_)
                return kc, sc_, new_lo, lo_bits

            kc, sc_, _, _ = lax.while_loop(
                ext_cond, ext_body, (kc, sc_, lo_bits0, hi_bits0))

            @pl.when(kc < _TOPK)
            def _pad():
                def fill(t, _):
                    j = t - kc
                    val = ext(suppv[pl.ds((j // _G) * _G, _G)], j % _G)
                    plsc.store_scatter(keepv, [spl(t)], spl(val),
                                       mask=lane0)
                    return _
                lax.fori_loop(kc, _TOPK, fill, 0)

            pltpu.sync_copy(keepv, out_hbm.at[wid])

    return sck


_make_sc = functools.cache(_make_sc)


@jax.jit
def kernel(pred_logits, pred_boxes, target_sizes):
    bs, nq, nc = pred_logits.shape
    prob = jax.nn.sigmoid(pred_logits).reshape(bs, nq * nc)
    prob_p = jnp.pad(prob, ((0, 0), (0, _NPT - nq * nc)))

    cx = pred_boxes[..., 0]; cy = pred_boxes[..., 1]
    w = pred_boxes[..., 2]; h = pred_boxes[..., 3]
    img_h = target_sizes[:, 0].astype(jnp.float32)
    img_w = target_sizes[:, 1].astype(jnp.float32)
    x1 = (cx - 0.5 * w) * img_w[:, None]
    y1 = (cy - 0.5 * h) * img_h[:, None]
    x2 = (cx + 0.5 * w) * img_w[:, None]
    y2 = (cy + 0.5 * h) * img_h[:, None]
    boxt = jnp.stack([x1, y1, x2, y2], axis=1)
    boxt = jnp.pad(boxt, ((0, 0), (0, 0), (0, _NQP - nq)))

    keep_gidx = _make_sc()(prob_p, boxt)

    keep = keep_gidx[:, :_TOPK]
    scores_out = jnp.take_along_axis(prob, keep, axis=1)
    labels_out = keep % nc
    q = keep // nc
    xy = jnp.stack([x1, y1, x2, y2], axis=-1)
    boxes_out = jnp.take_along_axis(xy, q[..., None], axis=1)
    return scores_out, labels_out, boxes_out
